# Initial kernel scaffold; baseline (speedup 1.0000x reference)
import jax
import jax.numpy as jnp
from jax.experimental import pallas as pl


def kernel(x, edge_index, batch, W1, a_src1, a_dst1, b1, W2, a_src2, a_dst2, b2, Wfc, bfc):
    raise NotImplementedError("write your pallas kernel here")


# speedup vs baseline: 5.1961x; 5.1961x over previous
import functools

import jax
import jax.numpy as jnp
from jax import lax
from jax.experimental import pallas as pl
from jax.experimental.pallas import tpu as pltpu
from jax.experimental.pallas import tpu_sc as plsc

N_ = 10000
NP_ = 10240
E_ = 320000
EA_ = E_ + N_
EP_ = 344064
EP128_ = EP_ // 128
T128_ = EP128_ // 16
NEG_SLOPE = 0.2
NG_ = 32

f32 = jnp.float32
i32 = jnp.int32


def _mm_layer1(xp, W1r, A1):
    BM = 512

    def body(x_ref, w_ref, a_ref, h_ref, al_ref):
        j = pl.program_id(1)
        h = jnp.dot(x_ref[...], w_ref[0], preferred_element_type=f32)
        h_ref[...] = h
        av = jnp.dot(h, a_ref[...], preferred_element_type=f32)

        @pl.when(j == 0)
        def _():
            al_ref[...] = av

        @pl.when(j > 0)
        def _():
            al_ref[...] = al_ref[...] + av

    return pl.pallas_call(
        body,
        grid=(20, 16),
        in_specs=[
            pl.BlockSpec((BM, 128), lambda i, j: (i, 0)),
            pl.BlockSpec((1, 128, 32), lambda i, j: (j, 0, 0)),
            pl.BlockSpec((32, 2), lambda i, j: (j, 0)),
        ],
        out_specs=[
            pl.BlockSpec((BM, 32), lambda i, j: (j * 20 + i, 0)),
            pl.BlockSpec((BM, 2), lambda i, j: (i, 0)),
        ],
        out_shape=[
            jax.ShapeDtypeStruct((16 * NP_, 32), f32),
            jax.ShapeDtypeStruct((NP_, 2), f32),
        ],
    )(xp, W1r, A1)


def _mm_layer2(agg1, den1t, b1, W2r, A2):
    BM = 512

    def body(agg0_ref, agg1_ref, agg2_ref, agg3_ref, den_ref, b_ref, w_ref,
             a_ref, h2_ref, al_ref):
        cblk = pl.program_id(1)
        k = pl.program_id(2)
        den = den_ref[0, :, 0] + den_ref[0, :, 1]
        inv = 2.0 / (den + 1e-30)
        aggs = [agg0_ref, agg1_ref, agg2_ref, agg3_ref]
        xk = jnp.concatenate(
            [
                jnp.maximum(
                    aggs[q][...] * inv[:, None]
                    + b_ref[q * 32:(q + 1) * 32][None, :],
                    0.0,
                )
                for q in range(4)
            ],
            axis=1,
        )
        contrib = jnp.dot(xk, w_ref[0], preferred_element_type=f32)

        @pl.when(k == 0)
        def _():
            h2_ref[...] = contrib

        @pl.when(k > 0)
        def _():
            h2_ref[...] = h2_ref[...] + contrib

        @pl.when(k == 3)
        def _():
            av = jnp.dot(h2_ref[...], a_ref[...], preferred_element_type=f32)

            @pl.when(cblk == 0)
            def _():
                al_ref[...] = av

            @pl.when(cblk > 0)
            def _():
                al_ref[...] = al_ref[...] + av

    return pl.pallas_call(
        body,
        grid=(20, 4, 4),
        in_specs=[
            pl.BlockSpec((BM, 32), lambda i, c, k: ((4 * k) * 20 + i, 0)),
            pl.BlockSpec((BM, 32), lambda i, c, k: ((4 * k + 1) * 20 + i, 0)),
            pl.BlockSpec((BM, 32), lambda i, c, k: ((4 * k + 2) * 20 + i, 0)),
            pl.BlockSpec((BM, 32), lambda i, c, k: ((4 * k + 3) * 20 + i, 0)),
            pl.BlockSpec((1, BM, 2), lambda i, c, k: (i, 0, 0)),
            pl.BlockSpec((128,), lambda i, c, k: (k,)),
            pl.BlockSpec((1, 128, 16), lambda i, c, k: (c, k, 0)),
            pl.BlockSpec((16, 2), lambda i, c, k: (c, 0)),
        ],
        out_specs=[
            pl.BlockSpec((BM, 16), lambda i, c, k: (c * 20 + i, 0)),
            pl.BlockSpec((BM, 2), lambda i, c, k: (i, 0)),
        ],
        out_shape=[
            jax.ShapeDtypeStruct((4 * NP_, 16), f32),
            jax.ShapeDtypeStruct((NP_, 2), f32),
        ],
    )(agg1, agg1, agg1, agg1, den1t, b1, W2r, A2)


def _pool_fc(agg2, den2, b2, batch, Wfc, bfc):
    def body(agg_ref, den_ref, b_ref, bat_ref, w_ref, bf_ref, o_ref):
        den = den_ref[0, 0:N_] + den_ref[1, 0:N_]
        inv = 2.0 / (den + 1e-30)
        b = bat_ref[...]
        oh = (lax.broadcasted_iota(i32, (NG_, N_), 0) == b[None, :]).astype(f32)
        parts = []
        for q in range(4):
            h2q = jnp.maximum(
                agg_ref[q * NP_:q * NP_ + N_, :] * inv[:, None]
                + b_ref[q * 16:(q + 1) * 16][None, :],
                0.0,
            )
            parts.append(jnp.dot(oh, h2q, preferred_element_type=f32))
        sums = jnp.concatenate(parts, axis=1)
        cnt = jnp.sum(oh, axis=1, keepdims=True)
        pooled = sums / jnp.maximum(cnt, 1.0)
        o_ref[...] = (
            jnp.dot(pooled, w_ref[...], preferred_element_type=f32)
            + bf_ref[...][None, :]
        )

    return pl.pallas_call(
        body,
        out_shape=jax.ShapeDtypeStruct((NG_, 2), f32),
    )(agg2, den2, b2, batch, Wfc, bfc)


def _make_edge_kernel(CH, CC, CHPS):
    mesh = plsc.VectorSubcoreMesh(
        core_axis_name="c", subcore_axis_name="s", num_cores=2, num_subcores=16
    )

    @functools.partial(
        pl.kernel,
        out_type=(
            jax.ShapeDtypeStruct((CH * NP_, CC), f32),
            jax.ShapeDtypeStruct((2 * NP_,), f32),
        ),
        mesh=mesh,
        compiler_params=pltpu.CompilerParams(
            needs_layout_passes=False, use_tc_tiling_on_sc=False
        ),
        scratch_types=[
            pltpu.VMEM((2 * NP_,), f32),
            pltpu.VMEM((T128_, 128), f32),
            pltpu.VMEM((T128_, 128), i32),
            pltpu.VMEM((T128_, 128), i32),
            pltpu.VMEM((128, CC), f32),
            pltpu.VMEM((80, CC), f32),
            pltpu.VMEM((640,), f32),
            pltpu.VMEM_SHARED((NP_, CC), f32),
            pltpu.VMEM_SHARED((NP_,), f32),
            pltpu.SemaphoreType.DMA,
        ],
    )
    def k(alpha_hbm, src_hbm, dst_hbm, h_hbm, agg_hbm, den_hbm,
          alpha_v, w_v, src_v, dst_v, rows_v, zb2, zb1, out_sh, den_sh, sem):
        s = lax.axis_index("s")
        c = lax.axis_index("c")
        zeros16 = jnp.zeros((16,), f32)
        iota16 = lax.broadcasted_iota(i32, (16,), 0)

        pltpu.sync_copy(alpha_hbm, alpha_v)
        pltpu.sync_copy(src_hbm.at[pl.ds(s * T128_, T128_)], src_v)
        pltpu.sync_copy(dst_hbm.at[pl.ds(s * T128_, T128_)], dst_v)

        @pl.loop(0, 40)
        def _(iz):
            zb1[pl.ds(iz * 16, 16)] = zeros16

        @pl.loop(0, 80)
        def _(iz):
            for kz in range(CC // 16):
                zb2[iz, pl.ds(kz * 16, 16)] = zeros16

        pltpu.sync_copy(zb1, den_sh.at[pl.ds(s * 640, 640)])
        plsc.subcore_barrier()

        @pl.loop(0, T128_)
        def _(jb):
            base = (s * T128_ + jb) * 128
            for kv in range(8):
                sidx = src_v[jb, pl.ds(kv * 16, 16)]
                didx = dst_v[jb, pl.ds(kv * 16, 16)]
                a_s = plsc.load_gather(alpha_v, [sidx * 2])
                a_d = plsc.load_gather(alpha_v, [didx * 2 + 1])
                e = a_s + a_d
                e = jnp.maximum(e, NEG_SLOPE * e)
                w = jnp.exp(e)
                ids = base + kv * 16 + iota16
                w = jnp.where(ids < EA_, w, 0.0)
                w_v[jb, pl.ds(kv * 16, 16)] = w

        @pl.loop(0, T128_)
        def _(jb):
            pltpu.sync_copy(w_v.at[jb], den_sh.at[dst_v.at[jb]], add=True)

        plsc.subcore_barrier()

        @pl.when(s == 0)
        def _():
            @pl.when(c == 0)
            def _():
                pltpu.sync_copy(den_sh, den_hbm.at[pl.ds(0, NP_)])

            @pl.when(c == 1)
            def _():
                pltpu.sync_copy(den_sh, den_hbm.at[pl.ds(NP_, NP_)])

        @pl.loop(0, T128_)
        def _(jb):
            for kv in range(8):
                sl = pl.ds(kv * 16, 16)
                src_v[jb, sl] = src_v[jb, sl] + c * (CHPS * NP_)

        for p in range(CHPS):
            if p > 0:
                @pl.loop(0, T128_)
                def _(jb):
                    for kv in range(8):
                        sl = pl.ds(kv * 16, 16)
                        src_v[jb, sl] = src_v[jb, sl] + NP_

            cc = c * CHPS + p

            @pl.loop(0, 8)
            def _(iz):
                pltpu.sync_copy(zb2, out_sh.at[pl.ds(s * 640 + iz * 80, 80)])

            plsc.subcore_barrier()

            @pl.loop(0, T128_)
            def _(jb):
                pltpu.async_copy(h_hbm.at[src_v.at[jb]], rows_v, sem).wait()

                @pl.loop(0, 128)
                def _(j):
                    wb = plsc.load_gather(
                        w_v, [jnp.full((16,), jb, i32), jnp.full((16,), j, i32)]
                    )
                    for kc in range(CC // 16):
                        sl = pl.ds(kc * 16, 16)
                        rows_v[j, sl] = rows_v[j, sl] * wb

                pltpu.sync_copy(rows_v, out_sh.at[dst_v.at[jb]], add=True)

            plsc.subcore_barrier()
            pltpu.sync_copy(
                out_sh.at[pl.ds(s * 640, 640)],
                agg_hbm.at[pl.ds(cc * NP_ + s * 640, 640)],
            )
            plsc.subcore_barrier()

    return k


_edge_l1 = _make_edge_kernel(16, 32, 8)
_edge_l2 = _make_edge_kernel(4, 16, 2)


def kernel(x, edge_index, batch, W1, a_src1, a_dst1, b1,
           W2, a_src2, a_dst2, b2, Wfc, bfc):
    loops = jnp.arange(N_, dtype=edge_index.dtype)
    src = jnp.concatenate([edge_index[0], loops])
    dst = jnp.concatenate([edge_index[1], loops])
    pad = EP_ - EA_
    src2d = jnp.pad(src, (0, pad)).reshape(EP128_, 128)
    dst2d = jnp.pad(dst, (0, pad)).reshape(EP128_, 128)
    xp = jnp.pad(x, ((0, NP_ - N_), (0, 0)))

    A1 = jnp.stack([a_src1, a_dst1], axis=1)
    A2 = jnp.stack([a_src2, a_dst2], axis=1)

    W1r = W1.reshape(128, 16, 32).transpose(1, 0, 2)
    h1, al1 = _mm_layer1(xp, W1r, A1)
    agg1, den1 = _edge_l1(al1.reshape(2 * NP_), src2d, dst2d, h1)
    den1t = den1.reshape(2, NP_).T.reshape(20, 512, 2)
    W2r = W2.reshape(512, 4, 16).transpose(1, 0, 2)
    h2, al2 = _mm_layer2(agg1, den1t, b1, W2r, A2)
    agg2, den2 = _edge_l2(al2.reshape(2 * NP_), src2d, dst2d, h2)
    return _pool_fc(agg2, den2.reshape(2, NP_), b2, batch, Wfc, bfc)


# speedup vs baseline: 7.9941x; 1.5385x over previous
import functools

import jax
import jax.numpy as jnp
from jax import lax
from jax.experimental import pallas as pl
from jax.experimental.pallas import tpu as pltpu
from jax.experimental.pallas import tpu_sc as plsc

N_ = 10000
NP_ = 10240
E_ = 320000
EA_ = E_ + N_
EP_ = 344064
EP128_ = EP_ // 128
T128_ = EP128_ // 16
NEG_SLOPE = 0.2
NG_ = 32

f32 = jnp.float32
i32 = jnp.int32


def _mm_layer1(xp, W1r, A1):
    BM = 512

    def body(x_ref, w_ref, a_ref, h_ref, al_ref):
        j = pl.program_id(1)
        h = jnp.dot(x_ref[...], w_ref[0], preferred_element_type=f32)
        h_ref[...] = h
        av = jnp.dot(h, a_ref[...], preferred_element_type=f32)

        @pl.when(j == 0)
        def _():
            al_ref[...] = av

        @pl.when(j > 0)
        def _():
            al_ref[...] = al_ref[...] + av

    return pl.pallas_call(
        body,
        grid=(20, 16),
        in_specs=[
            pl.BlockSpec((BM, 128), lambda i, j: (i, 0)),
            pl.BlockSpec((1, 128, 32), lambda i, j: (j, 0, 0)),
            pl.BlockSpec((32, 2), lambda i, j: (j, 0)),
        ],
        out_specs=[
            pl.BlockSpec((BM, 32), lambda i, j: (j * 20 + i, 0)),
            pl.BlockSpec((BM, 2), lambda i, j: (i, 0)),
        ],
        out_shape=[
            jax.ShapeDtypeStruct((16 * NP_, 32), f32),
            jax.ShapeDtypeStruct((NP_, 2), f32),
        ],
    )(xp, W1r, A1)


def _mm_layer2(agg1, den1t, b1, W2r, A2):
    BM = 512

    def body(agg0_ref, agg1_ref, agg2_ref, agg3_ref, den_ref, b_ref, w_ref,
             a_ref, h2_ref, al_ref):
        cblk = pl.program_id(1)
        k = pl.program_id(2)
        den = den_ref[0, :, 0] + den_ref[0, :, 1]
        inv = 2.0 / (den + 1e-30)
        aggs = [agg0_ref, agg1_ref, agg2_ref, agg3_ref]
        xk = jnp.concatenate(
            [
                jnp.maximum(
                    aggs[q][...] * inv[:, None]
                    + b_ref[q * 32:(q + 1) * 32][None, :],
                    0.0,
                )
                for q in range(4)
            ],
            axis=1,
        )
        contrib = jnp.dot(xk, w_ref[0], preferred_element_type=f32)

        @pl.when(k == 0)
        def _():
            h2_ref[...] = contrib

        @pl.when(k > 0)
        def _():
            h2_ref[...] = h2_ref[...] + contrib

        @pl.when(k == 3)
        def _():
            av = jnp.dot(h2_ref[...], a_ref[...], preferred_element_type=f32)

            @pl.when(cblk == 0)
            def _():
                al_ref[...] = av

            @pl.when(cblk > 0)
            def _():
                al_ref[...] = al_ref[...] + av

    return pl.pallas_call(
        body,
        grid=(20, 4, 4),
        in_specs=[
            pl.BlockSpec((BM, 32), lambda i, c, k: ((4 * k) * 20 + i, 0)),
            pl.BlockSpec((BM, 32), lambda i, c, k: ((4 * k + 1) * 20 + i, 0)),
            pl.BlockSpec((BM, 32), lambda i, c, k: ((4 * k + 2) * 20 + i, 0)),
            pl.BlockSpec((BM, 32), lambda i, c, k: ((4 * k + 3) * 20 + i, 0)),
            pl.BlockSpec((1, BM, 2), lambda i, c, k: (i, 0, 0)),
            pl.BlockSpec((128,), lambda i, c, k: (k,)),
            pl.BlockSpec((1, 128, 16), lambda i, c, k: (c, k, 0)),
            pl.BlockSpec((16, 2), lambda i, c, k: (c, 0)),
        ],
        out_specs=[
            pl.BlockSpec((BM, 16), lambda i, c, k: (c * 20 + i, 0)),
            pl.BlockSpec((BM, 2), lambda i, c, k: (i, 0)),
        ],
        out_shape=[
            jax.ShapeDtypeStruct((4 * NP_, 16), f32),
            jax.ShapeDtypeStruct((NP_, 2), f32),
        ],
    )(agg1, agg1, agg1, agg1, den1t, b1, W2r, A2)


def _pool_fc(agg2, den2, b2, batch, Wfc, bfc):
    def body(agg_ref, den_ref, b_ref, bat_ref, w_ref, bf_ref, o_ref):
        den = den_ref[0, 0:N_] + den_ref[1, 0:N_]
        inv = 2.0 / (den + 1e-30)
        b = bat_ref[...]
        oh = (lax.broadcasted_iota(i32, (NG_, N_), 0) == b[None, :]).astype(f32)
        parts = []
        for q in range(4):
            h2q = jnp.maximum(
                agg_ref[q * NP_:q * NP_ + N_, :] * inv[:, None]
                + b_ref[q * 16:(q + 1) * 16][None, :],
                0.0,
            )
            parts.append(jnp.dot(oh, h2q, preferred_element_type=f32))
        sums = jnp.concatenate(parts, axis=1)
        cnt = jnp.sum(oh, axis=1, keepdims=True)
        pooled = sums / jnp.maximum(cnt, 1.0)
        o_ref[...] = (
            jnp.dot(pooled, w_ref[...], preferred_element_type=f32)
            + bf_ref[...][None, :]
        )

    return pl.pallas_call(
        body,
        out_shape=jax.ShapeDtypeStruct((NG_, 2), f32),
    )(agg2, den2, b2, batch, Wfc, bfc)


def _make_edge_kernel(CH, CC, CHPS):
    mesh = plsc.VectorSubcoreMesh(
        core_axis_name="c", subcore_axis_name="s", num_cores=2, num_subcores=16
    )

    @functools.partial(
        pl.kernel,
        out_type=(
            jax.ShapeDtypeStruct((CH * NP_, CC), f32),
            jax.ShapeDtypeStruct((2 * NP_,), f32),
        ),
        mesh=mesh,
        compiler_params=pltpu.CompilerParams(
            needs_layout_passes=False, use_tc_tiling_on_sc=False
        ),
        scratch_types=[
            pltpu.VMEM((2 * NP_,), f32),
            pltpu.VMEM((T128_, 128), f32),
            pltpu.VMEM((T128_, 128), i32),
            pltpu.VMEM((T128_, 128), i32),
            pltpu.VMEM((128, CC), f32),
            pltpu.VMEM((128, CC), f32),
            pltpu.VMEM((80, CC), f32),
            pltpu.VMEM((640,), f32),
            pltpu.VMEM_SHARED((NP_, CC), f32),
            pltpu.VMEM_SHARED((NP_,), f32),
            pltpu.SemaphoreType.DMA,
            pltpu.SemaphoreType.DMA,
        ],
    )
    def k(alpha_hbm, src_hbm, dst_hbm, h_hbm, agg_hbm, den_hbm,
          alpha_v, w_v, src_v, dst_v, rows_a, rows_b, zb2, zb1, out_sh,
          den_sh, gsem, ssem):
        s = lax.axis_index("s")
        c = lax.axis_index("c")
        zeros16 = jnp.zeros((16,), f32)
        iota16 = lax.broadcasted_iota(i32, (16,), 0)

        pltpu.sync_copy(alpha_hbm, alpha_v)
        pltpu.sync_copy(src_hbm.at[pl.ds(s * T128_, T128_)], src_v)
        pltpu.sync_copy(dst_hbm.at[pl.ds(s * T128_, T128_)], dst_v)

        @pl.loop(0, 40)
        def _(iz):
            zb1[pl.ds(iz * 16, 16)] = zeros16

        @pl.loop(0, 80)
        def _(iz):
            for kz in range(CC // 16):
                zb2[iz, pl.ds(kz * 16, 16)] = zeros16

        pltpu.sync_copy(zb1, den_sh.at[pl.ds(s * 640, 640)])
        plsc.subcore_barrier()

        @pl.loop(0, T128_)
        def _(jb):
            base = (s * T128_ + jb) * 128
            for kv in range(8):
                sidx = src_v[jb, pl.ds(kv * 16, 16)]
                didx = dst_v[jb, pl.ds(kv * 16, 16)]
                a_s = plsc.load_gather(alpha_v, [sidx * 2])
                a_d = plsc.load_gather(alpha_v, [didx * 2 + 1])
                e = a_s + a_d
                e = jnp.maximum(e, NEG_SLOPE * e)
                w = jnp.exp(e)
                ids = base + kv * 16 + iota16
                w = jnp.where(ids < EA_, w, 0.0)
                w_v[jb, pl.ds(kv * 16, 16)] = w

        @pl.loop(0, T128_)
        def _(jb):
            pltpu.sync_copy(w_v.at[jb], den_sh.at[dst_v.at[jb]], add=True)

        plsc.subcore_barrier()

        @pl.when(s == 0)
        def _():
            @pl.when(c == 0)
            def _():
                pltpu.sync_copy(den_sh, den_hbm.at[pl.ds(0, NP_)])

            @pl.when(c == 1)
            def _():
                pltpu.sync_copy(den_sh, den_hbm.at[pl.ds(NP_, NP_)])

        @pl.loop(0, T128_)
        def _(jb):
            for kv in range(8):
                sl = pl.ds(kv * 16, 16)
                src_v[jb, sl] = src_v[jb, sl] + c * (CHPS * NP_)

        for p in range(CHPS):
            if p > 0:
                @pl.loop(0, T128_)
                def _(jb):
                    for kv in range(8):
                        sl = pl.ds(kv * 16, 16)
                        src_v[jb, sl] = src_v[jb, sl] + NP_

            cc = c * CHPS + p

            @pl.loop(0, 8)
            def _(iz):
                pltpu.sync_copy(zb2, out_sh.at[pl.ds(s * 640 + iz * 80, 80)])

            plsc.subcore_barrier()

            bufs = [rows_a, rows_b]
            pltpu.async_copy(h_hbm.at[src_v.at[0]], rows_a, gsem)

            @pl.loop(0, T128_ // 2)
            def _(g):
                for b in range(2):
                    jb = g * 2 + b
                    buf = bufs[b]
                    nbuf = bufs[1 - b]

                    @pl.when(jb >= 1)
                    def _():
                        pltpu.make_async_copy(
                            nbuf, out_sh.at[dst_v.at[jb]], ssem
                        ).wait()

                    @pl.when(jb <= T128_ - 2)
                    def _():
                        pltpu.async_copy(
                            h_hbm.at[src_v.at[jb + 1]], nbuf, gsem
                        )

                    pltpu.make_async_copy(
                        h_hbm.at[src_v.at[jb]], buf, gsem
                    ).wait()

                    @pl.loop(0, 128, unroll=4)
                    def _(j):
                        wb = plsc.load_gather(
                            w_v,
                            [jnp.full((16,), jb, i32), jnp.full((16,), j, i32)],
                        )
                        for kc in range(CC // 16):
                            sl = pl.ds(kc * 16, 16)
                            buf[j, sl] = buf[j, sl] * wb

                    pltpu.async_copy(
                        buf, out_sh.at[dst_v.at[jb]], ssem, add=True
                    )

            pltpu.make_async_copy(
                rows_b, out_sh.at[dst_v.at[T128_ - 1]], ssem
            ).wait()
            plsc.subcore_barrier()
            pltpu.sync_copy(
                out_sh.at[pl.ds(s * 640, 640)],
                agg_hbm.at[pl.ds(cc * NP_ + s * 640, 640)],
            )
            plsc.subcore_barrier()

    return k


_edge_l1 = _make_edge_kernel(16, 32, 8)
_edge_l2 = _make_edge_kernel(4, 16, 2)


def kernel(x, edge_index, batch, W1, a_src1, a_dst1, b1,
           W2, a_src2, a_dst2, b2, Wfc, bfc):
    loops = jnp.arange(N_, dtype=edge_index.dtype)
    src = jnp.concatenate([edge_index[0], loops])
    dst = jnp.concatenate([edge_index[1], loops])
    pad = EP_ - EA_
    src2d = jnp.pad(src, (0, pad)).reshape(EP128_, 128)
    dst2d = jnp.pad(dst, (0, pad)).reshape(EP128_, 128)
    xp = jnp.pad(x, ((0, NP_ - N_), (0, 0)))

    A1 = jnp.stack([a_src1, a_dst1], axis=1)
    A2 = jnp.stack([a_src2, a_dst2], axis=1)

    W1r = W1.reshape(128, 16, 32).transpose(1, 0, 2)
    h1, al1 = _mm_layer1(xp, W1r, A1)
    agg1, den1 = _edge_l1(al1.reshape(2 * NP_), src2d, dst2d, h1)
    den1t = den1.reshape(2, NP_).T.reshape(20, 512, 2)
    W2r = W2.reshape(512, 4, 16).transpose(1, 0, 2)
    h2, al2 = _mm_layer2(agg1, den1t, b1, W2r, A2)
    agg2, den2 = _edge_l2(al2.reshape(2 * NP_), src2d, dst2d, h2)
    return _pool_fc(agg2, den2.reshape(2, NP_), b2, batch, Wfc, bfc)
